# Initial kernel scaffold; baseline (speedup 1.0000x reference)
import jax
import jax.numpy as jnp
from jax.experimental import pallas as pl


def kernel(lidar_depth):
    raise NotImplementedError("write your pallas kernel here")


# speedup vs baseline: 1.6163x; 1.6163x over previous
import functools

import jax
import jax.numpy as jnp
from jax import lax
from jax.experimental import pallas as pl
from jax.experimental.pallas import tpu as pltpu
from jax.experimental.pallas import tpu_sc as plsc

DS = 16
D = 112
D_MIN = 2.0
D_INV_INT = 2.0
POS_MAX = 112.0 - 1e-06
SENTINEL = 100000.0

B, V, H, W = 8, 6, 256, 704
BV = B * V
HP = H // DS
WP = W // DS
NBANDS = BV * HP
NWORKERS = 32
BANDS_PER_W = NBANDS // NWORKERS
BAND_IN = DS * W
BAND_OUT = D * WP

GROUPS = ((0, 0), (16, 0), (28, 4))


def _sc_body(lidar, out, in_buf, out_buf):
    cid = lax.axis_index("c")
    sid = lax.axis_index("s")
    wid = sid * 2 + cid

    iota = lax.iota(jnp.int32, 16)
    zeros16 = jnp.zeros((16,), jnp.float32)
    sent = jnp.full((16,), SENTINEL, jnp.float32)

    def band_body(i, carry):
        b = wid * BANDS_PER_W + i

        pltpu.sync_copy(lidar.at[pl.ds(b * BAND_IN, BAND_IN)], in_buf)

        def zrow(r, c2):
            out_buf[pl.ds(r * 16, 16)] = zeros16
            return c2

        lax.fori_loop(0, BAND_OUT // 16, zrow, 0)

        for g_base, first_lane in GROUPS:
            col0 = (g_base + iota) * DS

            def rbody(r, acc, col0=col0):
                base = col0 + r * W
                for c in range(DS):
                    v = plsc.load_gather(in_buf, [base + c])
                    v = jnp.where(v == 0.0, SENTINEL, v)
                    acc = jnp.minimum(acc, v)
                return acc

            m = lax.fori_loop(0, DS, rbody, sent)

            pos = jnp.clip((m - D_MIN) * D_INV_INT, 0.0, POS_MAX)
            lower = pos.astype(jnp.int32)
            upper = jnp.minimum(lower + 1, D - 1)
            w_upper = jnp.clip(pos - lower.astype(jnp.float32), 0.0, 1.0)
            validf = jnp.where(m < SENTINEL, 1.0, 0.0)
            w_lower = (1.0 - w_upper) * validf
            w_upper = w_upper * validf

            ww = g_base + iota
            mask = None if first_lane == 0 else (iota >= first_lane)
            plsc.addupdate_scatter(out_buf, [lower * WP + ww], w_lower, mask=mask)
            plsc.addupdate_scatter(out_buf, [upper * WP + ww], w_upper, mask=mask)

        pltpu.sync_copy(out_buf, out.at[pl.ds(b * BAND_OUT, BAND_OUT)])
        return carry

    lax.fori_loop(0, BANDS_PER_W, band_body, 0)


@jax.jit
def kernel(lidar_depth):
    x = lidar_depth.reshape(-1)
    mesh = plsc.VectorSubcoreMesh(core_axis_name="c", subcore_axis_name="s")
    f = pl.kernel(
        _sc_body,
        out_type=jax.ShapeDtypeStruct((NBANDS * BAND_OUT,), jnp.float32),
        mesh=mesh,
        scratch_types=[
            pltpu.VMEM((BAND_IN,), jnp.float32),
            pltpu.VMEM((BAND_OUT,), jnp.float32),
        ],
        compiler_params=pltpu.CompilerParams(
            use_tc_tiling_on_sc=False, needs_layout_passes=False
        ),
    )
    y = f(x)
    return y.reshape(BV, HP, D, WP).transpose(0, 2, 1, 3)
